# baseline (speedup 1.0000x reference)
import dataclasses
import functools

import jax
import jax.numpy as jnp
from jax.experimental import pallas as pl
from jax.experimental.pallas import tpu as pltpu
from jax.experimental.pallas import tpu_sc as plsc

_GROUPS = (
    (0, 1, 2, 8, 25),
    (3, 24),
    (6, 7),
    (4, 9),
    (12, 13, 14, 15),
    (16, 17, 18, 19, 20, 21, 22, 23),
)

_N_COLS = 26
_N_OUT = 6
_LANES = 16
_TILE = 128
_SC_BLOCK_W = 1024
_TC_BLOCK_W = 65536
_TC_N_BLOCKS = 10


def _group_max_body(in_vmem, out_vmem):
    @plsc.parallel_loop(0, in_vmem.shape[1] // _LANES, unroll=8)
    def _(i):
        sl = pl.ds(i * _LANES, _LANES)
        for g, idx in enumerate(_GROUPS):
            m = in_vmem[idx[0], sl]
            for c in idx[1:]:
                m = jnp.maximum(m, in_vmem[c, sl])
            out_vmem[g, sl] = m


def _tc_body(x_ref, o_ref):
    for g, idx in enumerate(_GROUPS):
        m = x_ref[idx[0], :]
        for c in idx[1:]:
            m = jnp.maximum(m, x_ref[c, :])
        o_ref[g, :] = m


def _stitch_body(tc_ref, sc_ref, o_ref):
    del tc_ref
    o_ref[...] = sc_ref[...]


def _sc_run(act_t, n_rows, split):
    n_tiles = -(-n_rows // _TILE)
    sc_lanes = (n_tiles - split // _TILE) * _TILE
    sc_t0 = split // _TILE
    n_main = sc_lanes // _SC_BLOCK_W
    tail_t0 = sc_t0 + n_main * (_SC_BLOCK_W // _TILE)
    n_tail = n_tiles - tail_t0

    mesh = plsc.VectorSubcoreMesh(core_axis_name="c", subcore_axis_name="s")
    cp = pltpu.CompilerParams()
    if "needs_layout_passes" in pltpu.CompilerParams.__dataclass_fields__:
        cp = dataclasses.replace(cp, needs_layout_passes=False)
    if "use_tc_tiling_on_sc" in pltpu.CompilerParams.__dataclass_fields__:
        cp = dataclasses.replace(cp, use_tc_tiling_on_sc=True)

    @functools.partial(
        pl.kernel,
        out_type=jax.ShapeDtypeStruct((_N_OUT, sc_lanes), jnp.float32),
        mesh=mesh,
        compiler_params=cp,
    )
    def run(in_hbm, out_hbm):
        pltpu.emit_pipeline(
            _group_max_body,
            grid=(n_main,),
            in_specs=[
                pl.BlockSpec(
                    (_N_COLS, _SC_BLOCK_W),
                    lambda i: (0, i + sc_t0 // (_SC_BLOCK_W // _TILE)),
                )
            ],
            out_specs=[pl.BlockSpec((_N_OUT, _SC_BLOCK_W), lambda i: (0, i))],
            core_axis_name=("c", "s"),
            dimension_semantics=(pltpu.PARALLEL,),
        )(in_hbm, out_hbm)
        if n_tail:
            sc_nt = sc_lanes // _TILE - n_tail
            pltpu.emit_pipeline(
                _group_max_body,
                grid=(n_tail,),
                in_specs=[
                    pl.BlockSpec((_N_COLS, _TILE), lambda i: (0, i + tail_t0))
                ],
                out_specs=[
                    pl.BlockSpec((_N_OUT, _TILE), lambda i: (0, i + sc_nt))
                ],
                core_axis_name=("c", "s"),
                dimension_semantics=(pltpu.PARALLEL,),
            )(in_hbm, out_hbm)

    return run(act_t)


def kernel(activation):
    n_rows = activation.shape[0]
    act_t = activation.T
    split = _TC_N_BLOCKS * _TC_BLOCK_W

    sc_out = _sc_run(act_t, n_rows, split)

    tc_out = pl.pallas_call(
        _tc_body,
        grid=(_TC_N_BLOCKS,),
        in_specs=[pl.BlockSpec((_N_COLS, _TC_BLOCK_W), lambda i: (0, i))],
        out_specs=pl.BlockSpec((_N_OUT, _TC_BLOCK_W), lambda i: (0, i)),
        out_shape=jax.ShapeDtypeStruct((_N_OUT, n_rows), jnp.float32),
    )(act_t)

    sc_lanes = sc_out.shape[1]
    n_stitch = -(-sc_lanes // _TC_BLOCK_W)
    out_t = pl.pallas_call(
        _stitch_body,
        grid=(n_stitch,),
        in_specs=[
            pl.BlockSpec(
                (_N_OUT, _TC_BLOCK_W), lambda i: (0, i + _TC_N_BLOCKS)
            ),
            pl.BlockSpec((_N_OUT, _TC_BLOCK_W), lambda i: (0, i)),
        ],
        out_specs=pl.BlockSpec(
            (_N_OUT, _TC_BLOCK_W), lambda i: (0, i + _TC_N_BLOCKS)
        ),
        out_shape=jax.ShapeDtypeStruct((_N_OUT, n_rows), jnp.float32),
        input_output_aliases={0: 0},
    )(tc_out, sc_out)

    return out_t.T
